# Initial kernel scaffold; baseline (speedup 1.0000x reference)
import jax
import jax.numpy as jnp
from jax.experimental import pallas as pl


def kernel(context_features, realtime_back_category, realtime_goods, realtime_pair_click, realtime_passtime, realtime_user_group, goods_sparse, bucket_user_box_obj, bucket_goods_box_obj, bucket_goods_gross_obj, pair_feature, bucket_pair_box_obj, bucket_user_cspu_obj, bucket_ozid_cspu_obj, bucket_user_behavior_obj, cspu_idx, supplier_idx, lv2_idx, long_click, long_cart, long_buy, long_buy_level2, long_cart_level2, long_click_level2, short_click, short_cart, short_buy, short_click_level2, short_cart_level2, short_buy_level2, T_context, T_back_cat, T_goods_rt, T_pair_click, T_passtime, T_user_group, T_goods, T_bucket_user, T_bucket_goods, T_bucket_goods_gross, T_pair, T_bucket_pair, T_bucket_user_cspu, T_bucket_ozid_cspu, T_bucket_user_behavior, T_cspu, T_supplier, T_level2):
    raise NotImplementedError("write your pallas kernel here")


# speedup vs baseline: 2.3996x; 2.3996x over previous
import functools
import numpy as np
import jax
import jax.numpy as jnp
from jax import lax
from jax.experimental import pallas as pl
from jax.experimental.pallas import tpu as pltpu, tpu_sc as plsc

B = 1024
D = 32
NC, NS = 2, 16
NW = NC * NS
RPW = B // NW
CSZ = 128

FEATS = [
    (0, 5, 0), (1, 20, 1), (2, 20, 2), (3, 1, 3), (4, 20, 4), (5, 4, 5),
    (6, 26, 6), (7, 10, 7), (8, 10, 8), (9, 10, 9), (10, 10, 10),
    (11, 10, 11), (12, 10, 12), (13, 10, 13), (14, 10, 14),
    (15, 1, 15), (16, 1, 16), (17, 1, 17),
    (18, 50, 15), (19, 50, 15), (20, 50, 15),
    (21, 50, 17), (22, 50, 17), (23, 50, 17),
    (24, 20, 15), (25, 20, 15), (26, 20, 15),
    (27, 20, 17), (28, 20, 17), (29, 20, 17),
]
NCOLS = sum(n for _, n, _ in FEATS)

_off = 0
COL_OFF = []
for _, n, _ in FEATS:
    COL_OFF.append(_off)
    _off += n

TBL_FEATS = [[] for _ in range(18)]
for fi, (pos, n, t) in enumerate(FEATS):
    TBL_FEATS[t].append((pos, n, COL_OFF[fi]))
NT = [sum(n for _, n, _ in fs) for fs in TBL_FEATS]
CNT = [-(-RPW * nt // CSZ) for nt in NT]
CHOFF = np.concatenate([[0], np.cumsum(CNT)]).tolist()
CH = CHOFF[-1]


def _pad_worker_chunks(a):
    nt = a.shape[1]
    a = a.reshape(NW, RPW * nt)
    pad = -(-RPW * nt // CSZ) * CSZ - RPW * nt
    if pad:
        mod = jnp if isinstance(a, jax.Array) else np
        a = mod.pad(a, ((0, 0), (0, pad)), mode="edge")
    return a.reshape(NW, -1, CSZ)


def _dst_const():
    parts = []
    for t in range(18):
        cols = np.concatenate(
            [off + np.arange(n) for _, n, off in TBL_FEATS[t]])
        dst = (np.arange(B) * NCOLS)[:, None] + cols[None, :]
        parts.append(_pad_worker_chunks(dst.astype(np.int32)))
    return np.concatenate(parts, axis=1)


_DST = _dst_const()


@functools.cache
def _emb_kernel():
    mesh = plsc.VectorSubcoreMesh(core_axis_name="c", subcore_axis_name="s")

    @functools.partial(
        pl.kernel,
        out_type=jax.ShapeDtypeStruct((B * NCOLS, D), jnp.float32),
        mesh=mesh,
        compiler_params=pltpu.CompilerParams(use_tc_tiling_on_sc=False),
        scratch_types=[
            pltpu.VMEM((CH, CSZ), jnp.int32),
            pltpu.VMEM((CH, CSZ), jnp.int32),
            pltpu.VMEM((CSZ, D), jnp.float32),
            pltpu.SemaphoreType.DMA,
            pltpu.SemaphoreType.DMA,
        ],
    )
    def body(src_hbm, dst_hbm, *rest):
        tables = rest[:18]
        out_hbm = rest[18]
        src_v, dst_v, buf, sem_g, sem_s = rest[19:]
        w = lax.axis_index("s") * NC + lax.axis_index("c")
        pltpu.sync_copy(src_hbm.at[w], src_v)
        pltpu.sync_copy(dst_hbm.at[w], dst_v)
        for t in range(18):
            table = tables[t]
            base = CHOFF[t]

            def chunk(j, table=table, base=base):
                pltpu.async_copy(
                    table.at[src_v.at[base + j]], buf, sem_g).wait()
                pltpu.async_copy(
                    buf, out_hbm.at[dst_v.at[base + j]], sem_s).wait()

            pl.loop(0, CNT[t])(chunk)

    return body


def kernel(context_features, realtime_back_category, realtime_goods, realtime_pair_click, realtime_passtime, realtime_user_group, goods_sparse, bucket_user_box_obj, bucket_goods_box_obj, bucket_goods_gross_obj, pair_feature, bucket_pair_box_obj, bucket_user_cspu_obj, bucket_ozid_cspu_obj, bucket_user_behavior_obj, cspu_idx, supplier_idx, lv2_idx, long_click, long_cart, long_buy, long_buy_level2, long_cart_level2, long_click_level2, short_click, short_cart, short_buy, short_click_level2, short_cart_level2, short_buy_level2, T_context, T_back_cat, T_goods_rt, T_pair_click, T_passtime, T_user_group, T_goods, T_bucket_user, T_bucket_goods, T_bucket_goods_gross, T_pair, T_bucket_pair, T_bucket_user_cspu, T_bucket_ozid_cspu, T_bucket_user_behavior, T_cspu, T_supplier, T_level2):
    idxs = [context_features, realtime_back_category, realtime_goods, realtime_pair_click, realtime_passtime, realtime_user_group, goods_sparse, bucket_user_box_obj, bucket_goods_box_obj, bucket_goods_gross_obj, pair_feature, bucket_pair_box_obj, bucket_user_cspu_obj, bucket_ozid_cspu_obj, bucket_user_behavior_obj, cspu_idx, supplier_idx, lv2_idx, long_click, long_cart, long_buy, long_buy_level2, long_cart_level2, long_click_level2, short_click, short_cart, short_buy, short_click_level2, short_cart_level2, short_buy_level2]
    tables = [T_context, T_back_cat, T_goods_rt, T_pair_click, T_passtime, T_user_group, T_goods, T_bucket_user, T_bucket_goods, T_bucket_goods_gross, T_pair, T_bucket_pair, T_bucket_user_cspu, T_bucket_ozid_cspu, T_bucket_user_behavior, T_cspu, T_supplier, T_level2]

    parts = []
    for t in range(18):
        cat = jnp.concatenate(
            [idxs[pos].astype(jnp.int32).reshape(B, n)
             for pos, n, _ in TBL_FEATS[t]], axis=1)
        parts.append(_pad_worker_chunks(cat))
    src = jnp.concatenate(parts, axis=1)

    out = _emb_kernel()(src, jnp.asarray(_DST), *tables)
    return out.reshape(B, NCOLS * D)


# speedup vs baseline: 2.6131x; 1.0890x over previous
import functools
import numpy as np
import jax
import jax.numpy as jnp
from jax import lax
from jax.experimental import pallas as pl
from jax.experimental.pallas import tpu as pltpu, tpu_sc as plsc

B = 1024
D = 32
NC, NS = 2, 16
NW = NC * NS
RPW = B // NW
CSZ = 128
NBUF = 3

FEATS = [
    (0, 5, 0), (1, 20, 1), (2, 20, 2), (3, 1, 3), (4, 20, 4), (5, 4, 5),
    (6, 26, 6), (7, 10, 7), (8, 10, 8), (9, 10, 9), (10, 10, 10),
    (11, 10, 11), (12, 10, 12), (13, 10, 13), (14, 10, 14),
    (15, 1, 15), (16, 1, 16), (17, 1, 17),
    (18, 50, 15), (19, 50, 15), (20, 50, 15),
    (21, 50, 17), (22, 50, 17), (23, 50, 17),
    (24, 20, 15), (25, 20, 15), (26, 20, 15),
    (27, 20, 17), (28, 20, 17), (29, 20, 17),
]
NCOLS = sum(n for _, n, _ in FEATS)

_off = 0
COL_OFF = []
for _, n, _ in FEATS:
    COL_OFF.append(_off)
    _off += n

TBL_FEATS = [[] for _ in range(18)]
for fi, (pos, n, t) in enumerate(FEATS):
    TBL_FEATS[t].append((pos, n, COL_OFF[fi]))
NT = [sum(n for _, n, _ in fs) for fs in TBL_FEATS]
CNT = [-(-RPW * nt // CSZ) for nt in NT]
CHOFF = np.concatenate([[0], np.cumsum(CNT)]).tolist()
CH = CHOFF[-1]


def _pad_worker_chunks(a):
    nt = a.shape[1]
    a = a.reshape(NW, RPW * nt)
    pad = -(-RPW * nt // CSZ) * CSZ - RPW * nt
    if pad:
        mod = jnp if isinstance(a, jax.Array) else np
        a = mod.pad(a, ((0, 0), (0, pad)), mode="edge")
    return a.reshape(NW, -1, CSZ)


def _dst_const():
    parts = []
    for t in range(18):
        cols = np.concatenate(
            [off + np.arange(n) for _, n, off in TBL_FEATS[t]])
        dst = (np.arange(B) * NCOLS)[:, None] + cols[None, :]
        parts.append(_pad_worker_chunks(dst.astype(np.int32)))
    return np.concatenate(parts, axis=1)


_DST = _dst_const()


@functools.cache
def _emb_kernel():
    mesh = plsc.VectorSubcoreMesh(core_axis_name="c", subcore_axis_name="s")

    @functools.partial(
        pl.kernel,
        out_type=jax.ShapeDtypeStruct((B * NCOLS, D), jnp.float32),
        mesh=mesh,
        compiler_params=pltpu.CompilerParams(use_tc_tiling_on_sc=False),
        scratch_types=[
            pltpu.VMEM((CH, CSZ), jnp.int32),
            pltpu.VMEM((CH, CSZ), jnp.int32),
        ] + [pltpu.VMEM((CSZ, D), jnp.float32)] * NBUF
          + [pltpu.SemaphoreType.DMA] * (2 * NBUF),
    )
    def body(src_hbm, dst_hbm, *rest):
        tables = rest[:18]
        out_hbm = rest[18]
        src_v, dst_v = rest[19:21]
        bufs = rest[21:21 + NBUF]
        gsems = rest[21 + NBUF:21 + 2 * NBUF]
        ssems = rest[21 + 2 * NBUF:21 + 3 * NBUF]
        w = lax.axis_index("s") * NC + lax.axis_index("c")
        pltpu.sync_copy(src_hbm.at[w], src_v)
        pltpu.sync_copy(dst_hbm.at[w], dst_v)

        tbl_of = []
        for t in range(18):
            tbl_of += [t] * CNT[t]

        def gather(i):
            s = i % NBUF
            return pltpu.async_copy(
                tables[tbl_of[i]].at[src_v.at[i]], bufs[s], gsems[s])

        def scatter(i):
            s = i % NBUF
            return pltpu.async_copy(
                bufs[s], out_hbm.at[dst_v.at[i]], ssems[s])

        g = [None] * CH
        sc = [None] * CH
        for i in range(min(NBUF - 1, CH)):
            g[i] = gather(i)
        for i in range(CH):
            if i + NBUF - 1 < CH:
                if i >= 1:
                    sc[i - 1].wait()
                g[i + NBUF - 1] = gather(i + NBUF - 1)
            elif i >= 1:
                sc[i - 1].wait()
            g[i].wait()
            sc[i] = scatter(i)
        sc[CH - 1].wait()

    return body


def kernel(context_features, realtime_back_category, realtime_goods, realtime_pair_click, realtime_passtime, realtime_user_group, goods_sparse, bucket_user_box_obj, bucket_goods_box_obj, bucket_goods_gross_obj, pair_feature, bucket_pair_box_obj, bucket_user_cspu_obj, bucket_ozid_cspu_obj, bucket_user_behavior_obj, cspu_idx, supplier_idx, lv2_idx, long_click, long_cart, long_buy, long_buy_level2, long_cart_level2, long_click_level2, short_click, short_cart, short_buy, short_click_level2, short_cart_level2, short_buy_level2, T_context, T_back_cat, T_goods_rt, T_pair_click, T_passtime, T_user_group, T_goods, T_bucket_user, T_bucket_goods, T_bucket_goods_gross, T_pair, T_bucket_pair, T_bucket_user_cspu, T_bucket_ozid_cspu, T_bucket_user_behavior, T_cspu, T_supplier, T_level2):
    idxs = [context_features, realtime_back_category, realtime_goods, realtime_pair_click, realtime_passtime, realtime_user_group, goods_sparse, bucket_user_box_obj, bucket_goods_box_obj, bucket_goods_gross_obj, pair_feature, bucket_pair_box_obj, bucket_user_cspu_obj, bucket_ozid_cspu_obj, bucket_user_behavior_obj, cspu_idx, supplier_idx, lv2_idx, long_click, long_cart, long_buy, long_buy_level2, long_cart_level2, long_click_level2, short_click, short_cart, short_buy, short_click_level2, short_cart_level2, short_buy_level2]
    tables = [T_context, T_back_cat, T_goods_rt, T_pair_click, T_passtime, T_user_group, T_goods, T_bucket_user, T_bucket_goods, T_bucket_goods_gross, T_pair, T_bucket_pair, T_bucket_user_cspu, T_bucket_ozid_cspu, T_bucket_user_behavior, T_cspu, T_supplier, T_level2]

    parts = []
    for t in range(18):
        cat = jnp.concatenate(
            [idxs[pos].astype(jnp.int32).reshape(B, n)
             for pos, n, _ in TBL_FEATS[t]], axis=1)
        parts.append(_pad_worker_chunks(cat))
    src = jnp.concatenate(parts, axis=1)

    out = _emb_kernel()(src, jnp.asarray(_DST), *tables)
    return out.reshape(B, NCOLS * D)


# speedup vs baseline: 2.7238x; 1.0424x over previous
import functools
import numpy as np
import jax
import jax.numpy as jnp
from jax import lax
from jax.experimental import pallas as pl
from jax.experimental.pallas import tpu as pltpu, tpu_sc as plsc

B = 1024
D = 32
NC, NS = 2, 16
NW = NC * NS
RPW = B // NW
CSZ = 128
NBUF = 3

FEATS = [
    (0, 5, 0), (1, 20, 1), (2, 20, 2), (3, 1, 3), (4, 20, 4), (5, 4, 5),
    (6, 26, 6), (7, 10, 7), (8, 10, 8), (9, 10, 9), (10, 10, 10),
    (11, 10, 11), (12, 10, 12), (13, 10, 13), (14, 10, 14),
    (15, 1, 15), (16, 1, 16), (17, 1, 17),
    (18, 50, 15), (19, 50, 15), (20, 50, 15),
    (21, 50, 17), (22, 50, 17), (23, 50, 17),
    (24, 20, 15), (25, 20, 15), (26, 20, 15),
    (27, 20, 17), (28, 20, 17), (29, 20, 17),
]
NCOLS = sum(n for _, n, _ in FEATS)

_off = 0
COL_OFF = []
for _, n, _ in FEATS:
    COL_OFF.append(_off)
    _off += n

TBL_FEATS = [[] for _ in range(18)]
for fi, (pos, n, t) in enumerate(FEATS):
    TBL_FEATS[t].append((pos, n, COL_OFF[fi]))
NT = [sum(n for _, n, _ in fs) for fs in TBL_FEATS]
CNT = [-(-RPW * nt // CSZ) for nt in NT]

GROUP_B = [15]
GROUP_A = [t for t in range(18) if t not in GROUP_B]


def _pad_worker_chunks(a):
    nt = a.shape[1]
    a = a.reshape(NW, RPW * nt)
    pad = -(-RPW * nt // CSZ) * CSZ - RPW * nt
    if pad:
        mod = jnp if isinstance(a, jax.Array) else np
        a = mod.pad(a, ((0, 0), (0, pad)), mode="edge")
    return a.reshape(NW, -1, CSZ)


def _dst_part(t):
    cols = np.concatenate([off + np.arange(n) for _, n, off in TBL_FEATS[t]])
    dst = (np.arange(B) * NCOLS)[:, None] + cols[None, :]
    return _pad_worker_chunks(dst.astype(np.int32))


_DST = {g: np.concatenate([_dst_part(t) for t in grp], axis=1)
        for g, grp in (("A", GROUP_A), ("B", GROUP_B))}


def _pipeline(tables, out_hbm, src_v, dst_v, bufs, gsems, ssems, tbl_of):
    n = len(tbl_of)

    def gather(i):
        s = i % NBUF
        return pltpu.async_copy(
            tables[tbl_of[i]].at[src_v.at[i]], bufs[s], gsems[s])

    def scatter(i):
        s = i % NBUF
        return pltpu.async_copy(bufs[s], out_hbm.at[dst_v.at[i]], ssems[s])

    g = [None] * n
    sc = [None] * n
    for i in range(min(NBUF - 1, n)):
        g[i] = gather(i)
    for i in range(n):
        if i >= 1:
            sc[i - 1].wait()
        if i + NBUF - 1 < n:
            g[i + NBUF - 1] = gather(i + NBUF - 1)
        g[i].wait()
        sc[i] = scatter(i)
    sc[n - 1].wait()


def _make_kernel(group, produce_out):
    grp = GROUP_A if group == "A" else GROUP_B
    ntab = len(grp)
    tbl_of = []
    for k, t in enumerate(grp):
        tbl_of += [k] * CNT[t]
    ch = len(tbl_of)
    mesh = plsc.VectorSubcoreMesh(core_axis_name="c", subcore_axis_name="s")

    @functools.partial(
        pl.kernel,
        out_type=(jax.ShapeDtypeStruct((B * NCOLS, D), jnp.float32)
                  if produce_out else ()),
        mesh=mesh,
        compiler_params=pltpu.CompilerParams(use_tc_tiling_on_sc=False),
        scratch_types=[
            pltpu.VMEM((ch, CSZ), jnp.int32),
            pltpu.VMEM((ch, CSZ), jnp.int32),
        ] + [pltpu.VMEM((CSZ, D), jnp.float32)] * NBUF
          + [pltpu.SemaphoreType.DMA] * (2 * NBUF),
    )
    def body(src_hbm, dst_hbm, *rest):
        tables = rest[:ntab]
        out_hbm = rest[ntab]
        src_v, dst_v = rest[ntab + 1:ntab + 3]
        bufs = rest[ntab + 3:ntab + 3 + NBUF]
        gsems = rest[ntab + 3 + NBUF:ntab + 3 + 2 * NBUF]
        ssems = rest[ntab + 3 + 2 * NBUF:ntab + 3 + 3 * NBUF]
        w = lax.axis_index("s") * NC + lax.axis_index("c")
        pltpu.sync_copy(src_hbm.at[w], src_v)
        pltpu.sync_copy(dst_hbm.at[w], dst_v)
        _pipeline(tables, out_hbm, src_v, dst_v, bufs, gsems, ssems, tbl_of)

    return body


@functools.cache
def _kernels():
    return _make_kernel("A", True), _make_kernel("B", False)


def kernel(context_features, realtime_back_category, realtime_goods, realtime_pair_click, realtime_passtime, realtime_user_group, goods_sparse, bucket_user_box_obj, bucket_goods_box_obj, bucket_goods_gross_obj, pair_feature, bucket_pair_box_obj, bucket_user_cspu_obj, bucket_ozid_cspu_obj, bucket_user_behavior_obj, cspu_idx, supplier_idx, lv2_idx, long_click, long_cart, long_buy, long_buy_level2, long_cart_level2, long_click_level2, short_click, short_cart, short_buy, short_click_level2, short_cart_level2, short_buy_level2, T_context, T_back_cat, T_goods_rt, T_pair_click, T_passtime, T_user_group, T_goods, T_bucket_user, T_bucket_goods, T_bucket_goods_gross, T_pair, T_bucket_pair, T_bucket_user_cspu, T_bucket_ozid_cspu, T_bucket_user_behavior, T_cspu, T_supplier, T_level2):
    idxs = [context_features, realtime_back_category, realtime_goods, realtime_pair_click, realtime_passtime, realtime_user_group, goods_sparse, bucket_user_box_obj, bucket_goods_box_obj, bucket_goods_gross_obj, pair_feature, bucket_pair_box_obj, bucket_user_cspu_obj, bucket_ozid_cspu_obj, bucket_user_behavior_obj, cspu_idx, supplier_idx, lv2_idx, long_click, long_cart, long_buy, long_buy_level2, long_cart_level2, long_click_level2, short_click, short_cart, short_buy, short_click_level2, short_cart_level2, short_buy_level2]
    tables = [T_context, T_back_cat, T_goods_rt, T_pair_click, T_passtime, T_user_group, T_goods, T_bucket_user, T_bucket_goods, T_bucket_goods_gross, T_pair, T_bucket_pair, T_bucket_user_cspu, T_bucket_ozid_cspu, T_bucket_user_behavior, T_cspu, T_supplier, T_level2]

    def src_part(t):
        cat = jnp.concatenate(
            [idxs[pos].astype(jnp.int32).reshape(B, n)
             for pos, n, _ in TBL_FEATS[t]], axis=1)
        return _pad_worker_chunks(cat)

    src_a = jnp.concatenate([src_part(t) for t in GROUP_A], axis=1)
    src_b = jnp.concatenate([src_part(t) for t in GROUP_B], axis=1)

    ka, kb = _kernels()
    out_a = ka(src_a, jnp.asarray(_DST["A"]), *[tables[t] for t in GROUP_A])
    out_ref = jax.new_ref(out_a)
    kb(src_b, jnp.asarray(_DST["B"]), *[tables[t] for t in GROUP_B], out_ref)
    return out_ref[...].reshape(B, NCOLS * D)
